# Initial kernel scaffold; baseline (speedup 1.0000x reference)
import jax
import jax.numpy as jnp
from jax.experimental import pallas as pl


def kernel(x, edge_index, W1_l, b1_l, W1_r, W2_l, b2_l, W2_r):
    raise NotImplementedError("write your pallas kernel here")


# speedup vs baseline: 6.9950x; 6.9950x over previous
import functools

import jax
import jax.numpy as jnp
from jax import lax
from jax.experimental import pallas as pl
from jax.experimental.pallas import tpu as pltpu
from jax.experimental.pallas import tpu_sc as plsc

N = 10000
E = 320000
D = 128

NC = 2
NS = 16
NW = NC * NS
EPW = E // NW
CHUNK = 80
NCHUNK = EPW // CHUNK
NPAD = 10240
RPT = NPAD // NS
HALF = NPAD // 2
HR = HALF // 8

_mesh = plsc.VectorSubcoreMesh(core_axis_name="c", subcore_axis_name="s")


@functools.partial(
    pl.kernel,
    out_type=jax.ShapeDtypeStruct((NC, NPAD, D), jnp.float32),
    mesh=_mesh,
    compiler_params=pltpu.CompilerParams(needs_layout_passes=False),
    scratch_types=(
        pltpu.VMEM((NCHUNK, CHUNK), jnp.int32),
        pltpu.VMEM((NCHUNK, CHUNK), jnp.int32),
        pltpu.VMEM((CHUNK, D), jnp.float32),
        pltpu.VMEM((8, D), jnp.float32),
        pltpu.VMEM_SHARED((NPAD, D), jnp.float32),
        pltpu.SemaphoreType.DMA,
    ),
)
def _sc_aggregate(x_hbm, src_hbm, dst_hbm, agg_out,
                  srcb, dstb, rows_v, z8, agg_sh, sem):
    c = lax.axis_index("c")
    s = lax.axis_index("s")
    wid = c * NS + s

    zero16 = jnp.zeros((16,), jnp.float32)

    def zrow(i, _):
        def inner(j, _):
            z8[i, pl.ds(j * 16, 16)] = zero16
            return 0
        lax.fori_loop(0, D // 16, inner, 0)
        return 0
    lax.fori_loop(0, 8, zrow, 0)

    rbase = s * RPT
    def zshared(k, _):
        pltpu.sync_copy(z8, agg_sh.at[pl.ds(rbase + k * 8, 8)])
        return 0
    lax.fori_loop(0, RPT // 8, zshared, 0)

    pltpu.sync_copy(src_hbm.at[wid], srcb)
    pltpu.sync_copy(dst_hbm.at[wid], dstb)
    plsc.subcore_barrier()

    def body(ci, _):
        pltpu.async_copy(x_hbm.at[srcb.at[ci]], rows_v, sem).wait()
        pltpu.sync_copy(rows_v, agg_sh.at[dstb.at[ci]], add=True)
        return 0
    lax.fori_loop(0, NCHUNK, body, 0)

    plsc.subcore_barrier()

    pltpu.sync_copy(agg_sh.at[pl.ds(rbase, RPT)],
                    agg_out.at[c, pl.ds(rbase, RPT)])


@functools.partial(
    pl.kernel,
    out_type=jax.ShapeDtypeStruct((NC, NS, 2, HR, D), jnp.float32),
    mesh=_mesh,
    compiler_params=pltpu.CompilerParams(needs_layout_passes=False),
    scratch_types=(
        pltpu.VMEM((NCHUNK, CHUNK), jnp.int32),
        pltpu.VMEM((HR, D), jnp.float32),
    ),
)
def _sc_degree(dst_hbm, deg_out, dstb, hist):
    c = lax.axis_index("c")
    s = lax.axis_index("s")
    wid = c * NS + s

    zero16 = jnp.zeros((16,), jnp.float32)
    one16 = jnp.ones((16,), jnp.float32)
    lane = lax.iota(jnp.int32, 16)

    pltpu.sync_copy(dst_hbm.at[wid], dstb)

    for h in range(2):
        def zrow(i, _):
            def inner(j, _):
                hist[i, pl.ds(j * 16, 16)] = zero16
                return 0
            lax.fori_loop(0, D // 16, inner, 0)
            return 0
        lax.fori_loop(0, HR, zrow, 0)

        def body(ci, _):
            for g in range(CHUNK // 16):
                d16 = dstb[ci, pl.ds(g * 16, 16)]
                loc = d16 - h * HALF
                m = (loc >= 0) & (loc < HALF)
                locc = jnp.clip(loc, 0, HALF - 1)
                r = lax.shift_right_logical(locc, 3)
                col = lax.shift_left(jnp.bitwise_and(locc, 7), 4) + lane
                plsc.addupdate_scatter(hist, [r, col], one16, mask=m)
            return 0
        lax.fori_loop(0, NCHUNK, body, 0)

        pltpu.sync_copy(hist, deg_out.at[c, s, h])


BLK = 320
DR = BLK // 8


def _tc_layer_body(p_ref, d_ref, x_ref, wl_ref, b_ref, wr_ref, o_ref, *, last):
    a = p_ref[0] + p_ref[1]
    dblk = jnp.sum(d_ref[...], axis=(0, 1, 2))
    deg = jnp.sum(dblk.reshape(DR, 8, 16), axis=2).reshape(BLK, 1)
    mean = a / jnp.maximum(deg, 1.0)
    out = (jnp.dot(mean, wl_ref[...], preferred_element_type=jnp.float32)
           + b_ref[...]
           + jnp.dot(x_ref[...], wr_ref[...], preferred_element_type=jnp.float32))
    nrm = jnp.sqrt(jnp.sum(out * out, axis=1, keepdims=True))
    out = out / jnp.maximum(nrm, 1e-12)
    if last:
        m = jnp.max(out, axis=1, keepdims=True)
        t = out - m
        lse = jnp.log(jnp.sum(jnp.exp(t), axis=1, keepdims=True))
        o_ref[...] = t - lse
    else:
        o_ref[...] = jnp.maximum(out, 0.0)


def _tc_layer(p, degb, x, wl_t, b, wr_t, last):
    body = functools.partial(_tc_layer_body, last=last)
    return pl.pallas_call(
        body,
        grid=(NPAD // BLK,),
        in_specs=[
            pl.BlockSpec((NC, BLK, D), lambda i: (0, i, 0)),
            pl.BlockSpec((NC, NS, 1, DR, D), lambda i: (0, 0, i // NS, i % NS, 0)),
            pl.BlockSpec((BLK, D), lambda i: (i, 0)),
            pl.BlockSpec((D, D), lambda i: (0, 0)),
            pl.BlockSpec((1, D), lambda i: (0, 0)),
            pl.BlockSpec((D, D), lambda i: (0, 0)),
        ],
        out_specs=pl.BlockSpec((BLK, D), lambda i: (i, 0)),
        out_shape=jax.ShapeDtypeStruct((NPAD, D), jnp.float32),
    )(p, degb, x, wl_t, b, wr_t)


def kernel(x, edge_index, W1_l, b1_l, W1_r, W2_l, b2_l, W2_r):
    src3 = edge_index[0].reshape(NW, NCHUNK, CHUNK)
    dst3 = edge_index[1].reshape(NW, NCHUNK, CHUNK)
    xp = jnp.pad(x, ((0, NPAD - N), (0, 0)))

    agg1 = _sc_aggregate(xp, src3, dst3)
    degb = _sc_degree(dst3)
    h = _tc_layer(agg1, degb, xp, W1_l.T, b1_l.reshape(1, D), W1_r.T, last=False)

    agg2 = _sc_aggregate(h, src3, dst3)
    out = _tc_layer(agg2, degb, h, W2_l.T, b2_l.reshape(1, D), W2_r.T, last=True)
    return out[:N]
